# baseline (speedup 1.0000x reference)
import jax
import jax.numpy as jnp
from jax.experimental import pallas as pl
from jax.experimental.pallas import tpu as pltpu

_BM = 256


def _ln(h, g, b, eps=1e-5):
    m = jnp.mean(h, axis=-1, keepdims=True)
    c = h - m
    v = jnp.mean(c * c, axis=-1, keepdims=True)
    return c * jax.lax.rsqrt(v + eps) * g + b


def _mm(a, b):
    return jnp.dot(a, b, preferred_element_type=jnp.float32)


def _mm_bf16(a, b):
    return jnp.dot(a.astype(jnp.bfloat16), b.astype(jnp.bfloat16),
                   preferred_element_type=jnp.float32)


def _fused(adj_ref, x_ref, w0_ref, b0_ref, g0_ref, be0_ref, w1_ref,
           b1_ref, g1_ref, be1_ref, wl_ref, bl_ref, o_ref,
           s0_ref, s1_ref):
    i = pl.program_id(0)
    nb = pl.num_programs(0) // 2

    @pl.when(i == 0)
    def _prologue():
        s0_ref[...] = _mm(x_ref[...], w0_ref[...])

    @pl.when(i < nb)
    def _layer0():
        h = _mm_bf16(adj_ref[0], s0_ref[...]) + b0_ref[...]
        h = _ln(h, g0_ref[...], be0_ref[...])
        s1_ref[pl.ds(i * _BM, _BM), :] = _mm(h, w1_ref[...])

    @pl.when(i >= nb)
    def _layer1():
        h = _mm_bf16(adj_ref[0], s1_ref[...]) + b1_ref[...]
        h = _ln(h, g1_ref[...], be1_ref[...])
        h = jnp.where(h >= 0.0, h, 0.01 * h)
        y = _mm(h, wl_ref[...]) + bl_ref[...]
        m = jnp.max(y, axis=-1, keepdims=True)
        e = y - m
        o_ref[...] = e - jnp.log(jnp.sum(jnp.exp(e), axis=-1, keepdims=True))


def kernel(x, adj_matrices, W0, b0, g0, be0, W1, b1, g1, be1, Wl, bl):
    n, fan_in = x.shape
    fmid = W0.shape[1]
    fmid2 = W1.shape[1]
    fout = Wl.shape[1]
    nb = n // _BM

    full = lambda shp: pl.BlockSpec(shp, lambda i: (0,) * len(shp))
    row2 = lambda f: pl.BlockSpec((1, f), lambda i: (0, 0))
    adj_spec = pl.BlockSpec((1, _BM, n), lambda i: (i // nb, i % nb, 0))
    out_spec = pl.BlockSpec(
        (_BM, fout), lambda i: (jnp.where(i < nb, nb, i - nb), 0))

    out_padded = pl.pallas_call(
        _fused,
        grid=(2 * nb,),
        out_shape=jax.ShapeDtypeStruct((n + _BM, fout), jnp.float32),
        in_specs=[adj_spec, full((n, fan_in)), full((fan_in, fmid)),
                  row2(fmid), row2(fmid), row2(fmid),
                  full((fmid, fmid2)), row2(fmid2), row2(fmid2), row2(fmid2),
                  full((fmid2, fout)), row2(fout)],
        out_specs=out_spec,
        scratch_shapes=[pltpu.VMEM((n, fmid), jnp.float32),
                        pltpu.VMEM((n, fmid2), jnp.float32)],
        compiler_params=pltpu.CompilerParams(
            dimension_semantics=("arbitrary",)),
    )(adj_matrices, x, W0, b0.reshape(1, -1), g0.reshape(1, -1),
      be0.reshape(1, -1), W1, b1.reshape(1, -1), g1.reshape(1, -1),
      be1.reshape(1, -1), Wl, bl.reshape(1, -1))

    return out_padded[:n]


# speedup vs baseline: 1.0469x; 1.0469x over previous
import jax
import jax.numpy as jnp
from jax.experimental import pallas as pl
from jax.experimental.pallas import tpu as pltpu

_BM = 512


def _ln(h, g, b, eps=1e-5):
    m = jnp.mean(h, axis=-1, keepdims=True)
    c = h - m
    v = jnp.mean(c * c, axis=-1, keepdims=True)
    return c * jax.lax.rsqrt(v + eps) * g + b


def _mm(a, b):
    return jnp.dot(a, b, preferred_element_type=jnp.float32)


def _mm_bf16(a, b):
    return jnp.dot(a.astype(jnp.bfloat16), b.astype(jnp.bfloat16),
                   preferred_element_type=jnp.float32)


def _fused(adj_ref, x_ref, w0_ref, b0_ref, g0_ref, be0_ref, w1_ref,
           b1_ref, g1_ref, be1_ref, wl_ref, bl_ref, o_ref,
           s0_ref, s1_ref):
    i = pl.program_id(0)
    nb = pl.num_programs(0) // 2

    @pl.when(i == 0)
    def _prologue():
        s0_ref[...] = _mm(x_ref[...], w0_ref[...]).astype(jnp.bfloat16)

    @pl.when(i < nb)
    def _layer0():
        h = _mm_bf16(adj_ref[0], s0_ref[...]) + b0_ref[...]
        h = _ln(h, g0_ref[...], be0_ref[...])
        s1_ref[pl.ds(i * _BM, _BM), :] = _mm(
            h, w1_ref[...]).astype(jnp.bfloat16)

    @pl.when(i >= nb)
    def _layer1():
        h = _mm_bf16(adj_ref[0], s1_ref[...]) + b1_ref[...]
        h = _ln(h, g1_ref[...], be1_ref[...])
        h = jnp.where(h >= 0.0, h, 0.01 * h)
        y = _mm(h, wl_ref[...]) + bl_ref[...]
        m = jnp.max(y, axis=-1, keepdims=True)
        e = y - m
        o_ref[...] = e - jnp.log(jnp.sum(jnp.exp(e), axis=-1, keepdims=True))


def kernel(x, adj_matrices, W0, b0, g0, be0, W1, b1, g1, be1, Wl, bl):
    n, fan_in = x.shape
    fmid = W0.shape[1]
    fmid2 = W1.shape[1]
    fout = Wl.shape[1]
    nb = n // _BM

    full = lambda shp: pl.BlockSpec(shp, lambda i: (0,) * len(shp))
    row2 = lambda f: pl.BlockSpec((1, f), lambda i: (0, 0))
    adj_spec = pl.BlockSpec((1, _BM, n), lambda i: (i // nb, i % nb, 0))
    out_spec = pl.BlockSpec(
        (_BM, fout), lambda i: (jnp.where(i < nb, nb, i - nb), 0))

    out_padded = pl.pallas_call(
        _fused,
        grid=(2 * nb,),
        out_shape=jax.ShapeDtypeStruct((n + _BM, fout), jnp.float32),
        in_specs=[adj_spec, full((n, fan_in)), full((fan_in, fmid)),
                  row2(fmid), row2(fmid), row2(fmid),
                  full((fmid, fmid2)), row2(fmid2), row2(fmid2), row2(fmid2),
                  full((fmid2, fout)), row2(fout)],
        out_specs=out_spec,
        scratch_shapes=[pltpu.VMEM((n, fmid), jnp.bfloat16),
                        pltpu.VMEM((n, fmid2), jnp.bfloat16)],
        compiler_params=pltpu.CompilerParams(
            dimension_semantics=("arbitrary",)),
    )(adj_matrices, x, W0, b0.reshape(1, -1), g0.reshape(1, -1),
      be0.reshape(1, -1), W1, b1.reshape(1, -1), g1.reshape(1, -1),
      be1.reshape(1, -1), Wl, bl.reshape(1, -1))

    return out_padded[:n]
